# baseline (speedup 1.0000x reference)
import functools

import jax
import jax.numpy as jnp
import numpy as np
from jax import lax
from jax.experimental import pallas as pl
from jax.experimental.pallas import tpu as pltpu
from jax.experimental.pallas import tpu_sc as plsc

VOCAB = 1_000_000
EMBED = 64
HIDDEN = 64
BATCH = 16384

_INFO = plsc.get_sparse_core_info()
_NC = _INFO.num_cores
_NS = _INFO.num_subcores
_NW = _NC * _NS
_B_PER_W = BATCH // _NW
_L = 16
_RND = 256

_OFF = 262144
_BP = 8192
_NB = _OFF // _BP
_LAST = (VOCAB - 1) // _BP

_MASK = np.int32(-65536)


def _proj_body(tA_ref, tB_ref, tC_ref, tD_ref, w_ref, out_ref):
    dn = (((0,), (1,)), ((), ()))

    def proj_words(lo_ref, hi_ref):
        plo = jax.lax.dot_general(
            lo_ref[...], w_ref[...], dn, preferred_element_type=jnp.float32
        )
        phi = jax.lax.dot_general(
            hi_ref[...], w_ref[...], dn, preferred_element_type=jnp.float32
        )
        wlo = jax.lax.bitcast_convert_type(plo, jnp.uint32)
        whi = jax.lax.bitcast_convert_type(phi, jnp.uint32)
        return (whi & np.uint32(0xFFFF0000)) | (wlo >> 16)

    w01 = proj_words(tA_ref, tB_ref)
    w23 = proj_words(tC_ref, tD_ref)
    word = jnp.concatenate([w01, w23], axis=1)
    out_ref[...] = jax.lax.bitcast_convert_type(word, jnp.int32)


def _tc_project_table(tableT, w):
    return pl.pallas_call(
        _proj_body,
        grid=(_NB,),
        in_specs=[
            pl.BlockSpec((EMBED, _BP), lambda i: (0, i)),
            pl.BlockSpec((EMBED, _BP), lambda i: (0, i + _NB)),
            pl.BlockSpec((EMBED, _BP), lambda i: (0, i + 2 * _NB)),
            pl.BlockSpec(
                (EMBED, _BP), lambda i: (0, jnp.minimum(i + 3 * _NB, _LAST))
            ),
            pl.BlockSpec((HIDDEN, EMBED), lambda i: (0, 0)),
        ],
        out_specs=pl.BlockSpec((_BP, 2 * HIDDEN), lambda i: (i, 0)),
        out_shape=jax.ShapeDtypeStruct((_OFF, 2 * HIDDEN), jnp.int32),
        compiler_params=pltpu.CompilerParams(
            fuse_transposed_lhs_in_matmul=True
        ),
    )(tableT, tableT, tableT, tableT, w)


def _gather_body(idx_hbm, pair4_hbm, out_hbm, idx_v, rows_v, sem):
    wid = lax.axis_index("s") * _NC + lax.axis_index("c")
    base = wid * _B_PER_W
    pltpu.sync_copy(idx_hbm.at[pl.ds(base, _B_PER_W)], idx_v)

    for h in range(_B_PER_W // _RND):

        def fetch(g, _):
            v = idx_v[pl.ds(h * _RND + g * _L, _L)]
            for l in range(_L):
                p = v[l] & (_OFF - 1)
                pltpu.async_copy(pair4_hbm.at[p], rows_v.at[g * _L + l], sem)
            return 0

        lax.fori_loop(0, _RND // _L, fetch, 0)
        pltpu.make_async_copy(
            pair4_hbm.at[pl.ds(0, _RND)], rows_v, sem
        ).wait()
        pltpu.sync_copy(rows_v, out_hbm.at[pl.ds(base + h * _RND, _RND)])


@functools.partial(
    pl.kernel,
    mesh=plsc.VectorSubcoreMesh(core_axis_name="c", subcore_axis_name="s"),
    out_type=jax.ShapeDtypeStruct((BATCH, 2 * HIDDEN), jnp.int32),
    scratch_types=[
        pltpu.VMEM((_B_PER_W,), jnp.int32),
        pltpu.VMEM((_RND, 2 * HIDDEN), jnp.int32),
        pltpu.SemaphoreType.DMA,
    ],
)
def _sc_gather(idx_hbm, pair4_hbm, out_hbm, idx_v, rows_v, sem):
    _gather_body(idx_hbm, pair4_hbm, out_hbm, idx_v, rows_v, sem)


def _sel_body(w_ref, idx_ref, b_ref, out_ref):
    w = w_ref[...]
    q = idx_ref[...] >> 18
    w64 = jnp.where(q >= 2, w[:, HIDDEN:], w[:, :HIDDEN])
    sel = jnp.where((q & 1) == 1, w64 & _MASK, w64 << 16)
    f = jax.lax.bitcast_convert_type(sel, jnp.float32) + b_ref[...]
    out_ref[...] = f.T


_BS = 2048


def _tc_select(words, idx2d, brow):
    return pl.pallas_call(
        _sel_body,
        grid=(BATCH // _BS,),
        in_specs=[
            pl.BlockSpec((_BS, 2 * HIDDEN), lambda i: (i, 0)),
            pl.BlockSpec((_BS, 1), lambda i: (i, 0)),
            pl.BlockSpec((1, HIDDEN), lambda i: (0, 0)),
        ],
        out_specs=pl.BlockSpec((HIDDEN, _BS), lambda i: (0, i)),
        out_shape=jax.ShapeDtypeStruct((HIDDEN, BATCH), jnp.float32),
    )(words, idx2d, brow)


def kernel(user_embeds, table, W, b):
    idx = user_embeds.astype(jnp.int32)
    pair4 = _tc_project_table(table.T, W)
    words = _sc_gather(idx, pair4)
    outT = _tc_select(words, idx.reshape(BATCH, 1), b.reshape(1, HIDDEN))
    return outT.T


# speedup vs baseline: 1.0573x; 1.0573x over previous
import functools

import jax
import jax.numpy as jnp
import numpy as np
from jax import lax
from jax.experimental import pallas as pl
from jax.experimental.pallas import tpu as pltpu
from jax.experimental.pallas import tpu_sc as plsc

VOCAB = 1_000_000
EMBED = 64
HIDDEN = 64
BATCH = 16384

_INFO = plsc.get_sparse_core_info()
_NC = _INFO.num_cores
_NS = _INFO.num_subcores
_NW = _NC * _NS
_B_PER_W = BATCH // _NW
_L = 16
_RND = 128

_BP = 16384
_NBLK = 31
_OFF = _NBLK * _BP


def _proj_body(tA_ref, tB_ref, w_ref, out_ref):
    dn = (((0,), (1,)), ((), ()))
    pA = jax.lax.dot_general(
        tA_ref[...], w_ref[...], dn, preferred_element_type=jnp.float32
    )
    pB = jax.lax.dot_general(
        tB_ref[...], w_ref[...], dn, preferred_element_type=jnp.float32
    )
    wa = jax.lax.bitcast_convert_type(pA, jnp.uint32)
    wb = jax.lax.bitcast_convert_type(pB, jnp.uint32)
    word = (wb & np.uint32(0xFFFF0000)) | (wa >> 16)
    out_ref[...] = jax.lax.bitcast_convert_type(word, jnp.int32)


def _tc_project_table(tableT, w):
    return pl.pallas_call(
        _proj_body,
        grid=(_NBLK,),
        in_specs=[
            pl.BlockSpec((EMBED, _BP), lambda i: (0, i)),
            pl.BlockSpec(
                (EMBED, _BP),
                lambda i: (0, jnp.minimum(i + _NBLK, (VOCAB - 1) // _BP)),
            ),
            pl.BlockSpec((HIDDEN, EMBED), lambda i: (0, 0)),
        ],
        out_specs=pl.BlockSpec((_BP, HIDDEN), lambda i: (i, 0)),
        out_shape=jax.ShapeDtypeStruct((_OFF, HIDDEN), jnp.int32),
        compiler_params=pltpu.CompilerParams(
            fuse_transposed_lhs_in_matmul=True
        ),
    )(tableT, tableT, w)


_MASK = np.int32(-65536)


def _gather_body(idx_hbm, pair2_hbm, out_hbm, idx_v, rows_v, sem):
    wid = lax.axis_index("s") * _NC + lax.axis_index("c")
    base = wid * _B_PER_W
    pltpu.sync_copy(idx_hbm.at[pl.ds(base, _B_PER_W)], idx_v)

    def fetch(g, _):
        v = idx_v[pl.ds(g * _L, _L)]
        for l in range(_L):
            iv = v[l]
            p = jnp.where(iv >= _OFF, iv - _OFF, iv)
            pltpu.async_copy(pair2_hbm.at[p], rows_v.at[g * _L + l], sem)
        return 0

    lax.fori_loop(0, _B_PER_W // _L, fetch, 0)
    pltpu.make_async_copy(
        pair2_hbm.at[pl.ds(0, _B_PER_W)], rows_v, sem
    ).wait()
    pltpu.sync_copy(rows_v, out_hbm.at[pl.ds(base, _B_PER_W)])


@functools.partial(
    pl.kernel,
    mesh=plsc.VectorSubcoreMesh(core_axis_name="c", subcore_axis_name="s"),
    out_type=jax.ShapeDtypeStruct((BATCH, HIDDEN), jnp.int32),
    scratch_types=[
        pltpu.VMEM((_B_PER_W,), jnp.int32),
        pltpu.VMEM((_B_PER_W, HIDDEN), jnp.int32),
        pltpu.SemaphoreType.DMA,
    ],
)
def _sc_gather(idx_hbm, pair2_hbm, out_hbm, idx_v, rows_v, sem):
    _gather_body(idx_hbm, pair2_hbm, out_hbm, idx_v, rows_v, sem)


def _sel_body(w_ref, idx_ref, b_ref, out_ref):
    w = w_ref[...]
    hi = idx_ref[...] >= _OFF
    sel = jnp.where(hi, w & _MASK, w << 16)
    f = jax.lax.bitcast_convert_type(sel, jnp.float32) + b_ref[...]
    out_ref[...] = f.T


_BS = 4096


def _tc_select(words, idx2d, brow):
    return pl.pallas_call(
        _sel_body,
        grid=(BATCH // _BS,),
        in_specs=[
            pl.BlockSpec((_BS, HIDDEN), lambda i: (i, 0)),
            pl.BlockSpec((_BS, 1), lambda i: (i, 0)),
            pl.BlockSpec((1, HIDDEN), lambda i: (0, 0)),
        ],
        out_specs=pl.BlockSpec((HIDDEN, _BS), lambda i: (0, i)),
        out_shape=jax.ShapeDtypeStruct((HIDDEN, BATCH), jnp.float32),
    )(words, idx2d, brow)


def kernel(user_embeds, table, W, b):
    idx = user_embeds.astype(jnp.int32)
    pair2 = _tc_project_table(table.T, W)
    words = _sc_gather(idx, pair2)
    outT = _tc_select(words, idx.reshape(BATCH, 1), b.reshape(1, HIDDEN))
    return outT.T


# speedup vs baseline: 1.0626x; 1.0050x over previous
import functools

import jax
import jax.numpy as jnp
import numpy as np
from jax import lax
from jax.experimental import pallas as pl
from jax.experimental.pallas import tpu as pltpu
from jax.experimental.pallas import tpu_sc as plsc

VOCAB = 1_000_000
EMBED = 64
HIDDEN = 64
BATCH = 16384

_INFO = plsc.get_sparse_core_info()
_NC = _INFO.num_cores
_NS = _INFO.num_subcores
_NW = _NC * _NS
_B_PER_W = BATCH // _NW
_L = 16

_BP = 16384
_NBLK = 31
_OFF = _NBLK * _BP


def _proj_body(tA_ref, tB_ref, w_ref, out_ref):
    dn = (((0,), (1,)), ((), ()))
    pA = jax.lax.dot_general(
        tA_ref[...], w_ref[...], dn, preferred_element_type=jnp.float32
    )
    pB = jax.lax.dot_general(
        tB_ref[...], w_ref[...], dn, preferred_element_type=jnp.float32
    )
    wa = jax.lax.bitcast_convert_type(pA, jnp.uint32)
    wb = jax.lax.bitcast_convert_type(pB, jnp.uint32)
    word = (wb & np.uint32(0xFFFF0000)) | (wa >> 16)
    out_ref[...] = jax.lax.bitcast_convert_type(word, jnp.int32)


def _tc_project_table(tableT, w):
    return pl.pallas_call(
        _proj_body,
        grid=(_NBLK,),
        in_specs=[
            pl.BlockSpec((EMBED, _BP), lambda i: (0, i)),
            pl.BlockSpec(
                (EMBED, _BP),
                lambda i: (0, jnp.minimum(i + _NBLK, (VOCAB - 1) // _BP)),
            ),
            pl.BlockSpec((HIDDEN, EMBED), lambda i: (0, 0)),
        ],
        out_specs=pl.BlockSpec((_BP, HIDDEN), lambda i: (i, 0)),
        out_shape=jax.ShapeDtypeStruct((_OFF, HIDDEN), jnp.int32),
        compiler_params=pltpu.CompilerParams(
            fuse_transposed_lhs_in_matmul=True
        ),
    )(tableT, tableT, w)


_MASK = np.int32(-65536)


def _gather_body(idx_hbm, pair2_hbm, out_hbm, idx_v, rows_v, sem):
    wid = lax.axis_index("s") * _NC + lax.axis_index("c")
    base = wid * _B_PER_W
    pltpu.sync_copy(idx_hbm.at[pl.ds(base, _B_PER_W)], idx_v)

    def fetch(g, _):
        v = idx_v[pl.ds(g * _L, _L)]
        for l in range(_L):
            iv = v[l]
            p = jnp.where(iv >= _OFF, iv - _OFF, iv)
            pltpu.async_copy(pair2_hbm.at[p], rows_v.at[g * _L + l], sem)
        return 0

    lax.fori_loop(0, _B_PER_W // _L, fetch, 0)
    pltpu.make_async_copy(
        pair2_hbm.at[pl.ds(0, _B_PER_W)], rows_v, sem
    ).wait()
    pltpu.sync_copy(rows_v, out_hbm.at[pl.ds(base, _B_PER_W)])


@functools.partial(
    pl.kernel,
    mesh=plsc.VectorSubcoreMesh(core_axis_name="c", subcore_axis_name="s"),
    out_type=jax.ShapeDtypeStruct((BATCH, HIDDEN), jnp.int32),
    scratch_types=[
        pltpu.VMEM((_B_PER_W,), jnp.int32),
        pltpu.VMEM((_B_PER_W, HIDDEN), jnp.int32),
        pltpu.SemaphoreType.DMA,
    ],
)
def _sc_gather(idx_hbm, pair2_hbm, out_hbm, idx_v, rows_v, sem):
    _gather_body(idx_hbm, pair2_hbm, out_hbm, idx_v, rows_v, sem)


def _sel_body(w_ref, idx_ref, b_ref, out_ref):
    w = w_ref[...]
    hi = idx_ref[...] >= _OFF
    sel = jnp.where(hi, w & _MASK, w << 16)
    f = jax.lax.bitcast_convert_type(sel, jnp.float32) + b_ref[...]
    out_ref[...] = f.T


_BS = 4096


def _tc_select(words, idx2d, brow):
    return pl.pallas_call(
        _sel_body,
        grid=(BATCH // _BS,),
        in_specs=[
            pl.BlockSpec((_BS, HIDDEN), lambda i: (i, 0)),
            pl.BlockSpec((_BS, 1), lambda i: (i, 0)),
            pl.BlockSpec((1, HIDDEN), lambda i: (0, 0)),
        ],
        out_specs=pl.BlockSpec((HIDDEN, _BS), lambda i: (0, i)),
        out_shape=jax.ShapeDtypeStruct((HIDDEN, BATCH), jnp.float32),
    )(words, idx2d, brow)


def kernel(user_embeds, table, W, b):
    idx = user_embeds.astype(jnp.int32)
    pair2 = _tc_project_table(table.T, W)
    words = _sc_gather(idx, pair2)
    outT = _tc_select(words, idx.reshape(BATCH, 1), b.reshape(1, HIDDEN))
    return outT.T
